# baseline (speedup 1.0000x reference)
import functools

import jax
import jax.numpy as jnp
from jax import lax
from jax.experimental import pallas as pl
from jax.experimental.pallas import tpu as pltpu
from jax.experimental.pallas import tpu_sc as plsc

NORM = 1.0
MEAN = 0.0
EMBED_DIM = 64
NUM_WORKERS = 32
CHUNK = 128


def _make_gather(total_rows: int):
    assert total_rows % (NUM_WORKERS * CHUNK) == 0
    per_worker = total_rows // NUM_WORKERS
    n_chunks = per_worker // CHUNK
    mesh = plsc.VectorSubcoreMesh(core_axis_name="c", subcore_axis_name="s")

    @functools.partial(
        pl.kernel,
        out_type=jax.ShapeDtypeStruct((total_rows, EMBED_DIM), jnp.float32),
        mesh=mesh,
        scratch_types=[
            pltpu.VMEM((per_worker,), jnp.int32),
            pltpu.VMEM((CHUNK, EMBED_DIM), jnp.float32),
            pltpu.VMEM((CHUNK, EMBED_DIM), jnp.float32),
            pltpu.SemaphoreType.DMA,
            pltpu.SemaphoreType.DMA,
        ],
        compiler_params=pltpu.CompilerParams(use_tc_tiling_on_sc=False),
    )
    def gather_kernel(table_hbm, idx_hbm, out_hbm, idx_v, rows_a, rows_b, sem_a, sem_b):
        wid = lax.axis_index("s") * 2 + lax.axis_index("c")
        base = wid * per_worker
        pltpu.sync_copy(idx_hbm.at[pl.ds(base, per_worker)], idx_v)

        rows = (rows_a, rows_b)
        sems = (sem_a, sem_b)

        def start(j, b):
            off = pl.multiple_of(j * CHUNK, CHUNK)
            pltpu.async_copy(table_hbm.at[idx_v.at[pl.ds(off, CHUNK)]], rows[b], sems[b])

        def drain(j, b):
            off = pl.multiple_of(j * CHUNK, CHUNK)
            pltpu.make_async_copy(table_hbm.at[idx_v.at[pl.ds(off, CHUNK)]], rows[b], sems[b]).wait()
            pltpu.sync_copy(rows[b], out_hbm.at[pl.ds(base + off, CHUNK)])

        start(0, 0)

        def body(j, _):
            def step(bb):
                start(j + 1, 1 - bb)
                drain(j, bb)

            lax.cond(lax.rem(j, 2) == 0, lambda: step(0), lambda: step(1))
            return 0

        lax.fori_loop(0, n_chunks - 1, body, 0)
        last = n_chunks - 1
        lax.cond(lax.rem(last, 2) == 0, lambda: drain(last, 0), lambda: drain(last, 1))

    return gather_kernel


def kernel(vars_, index):
    b, s = index.shape
    idx = index.reshape(-1).astype(jnp.int32)
    out = _make_gather(idx.shape[0])(vars_, idx)
    return out.reshape(b, s, EMBED_DIM)
